# Initial kernel scaffold; baseline (speedup 1.0000x reference)
import jax
import jax.numpy as jnp
from jax.experimental import pallas as pl


def kernel(x, edge_index, Wl, bl, Wr, br, att, bias):
    raise NotImplementedError("write your pallas kernel here")


# speedup vs baseline: 1.0424x; 1.0424x over previous
import functools

import jax
import jax.numpy as jnp
from jax.experimental import pallas as pl
from jax.experimental.pallas import tpu as pltpu

N = 50000
E = 800000
H = 6
C = 16
D = H * C

_BLK = 2000


def _proj_body(x_ref, wlT_ref, bl_ref, wrT_ref, br_ref, xl_ref, xr_ref):
    xb = x_ref[...]
    xl_ref[...] = xb @ wlT_ref[...] + bl_ref[...][None, :]
    xr_ref[...] = xb @ wrT_ref[...] + br_ref[...][None, :]


def _finish_body(x_ref, num_ref, den_ref, bias_ref, y_ref):
    num = num_ref[...].reshape(_BLK, H, C)
    den = den_ref[...].reshape(_BLK, H, 1)
    out = (num / den).reshape(_BLK, D) + bias_ref[...][None, :]
    y_ref[...] = jnp.maximum(x_ref[...] + out, 0.0)


def kernel(x, edge_index, Wl, bl, Wr, br, att, bias):
    grid = N // _BLK
    xl, xr = pl.pallas_call(
        _proj_body,
        grid=(grid,),
        in_specs=[
            pl.BlockSpec((_BLK, D), lambda i: (i, 0)),
            pl.BlockSpec((D, D), lambda i: (0, 0)),
            pl.BlockSpec((D,), lambda i: (0,)),
            pl.BlockSpec((D, D), lambda i: (0, 0)),
            pl.BlockSpec((D,), lambda i: (0,)),
        ],
        out_specs=[
            pl.BlockSpec((_BLK, D), lambda i: (i, 0)),
            pl.BlockSpec((_BLK, D), lambda i: (i, 0)),
        ],
        out_shape=[
            jax.ShapeDtypeStruct((N, D), jnp.float32),
            jax.ShapeDtypeStruct((N, D), jnp.float32),
        ],
    )(x, Wl.T, bl, Wr.T, br)

    loops = jnp.arange(N, dtype=edge_index.dtype)
    src = jnp.concatenate([edge_index[0], loops])
    dst = jnp.concatenate([edge_index[1], loops])
    xl3 = xl.reshape(N, H, C)
    xr3 = xr.reshape(N, H, C)
    m = xl3[src] + xr3[dst]
    m = jnp.where(m > 0, m, 0.2 * m)
    alpha = jnp.exp(jnp.sum(m * att, axis=-1))
    den = jax.ops.segment_sum(alpha, dst, num_segments=N)
    num = jax.ops.segment_sum(
        xl3[src] * alpha[:, :, None], dst, num_segments=N
    ).reshape(N, D)

    y = pl.pallas_call(
        _finish_body,
        grid=(grid,),
        in_specs=[
            pl.BlockSpec((_BLK, D), lambda i: (i, 0)),
            pl.BlockSpec((_BLK, D), lambda i: (i, 0)),
            pl.BlockSpec((_BLK, H), lambda i: (i, 0)),
            pl.BlockSpec((D,), lambda i: (0,)),
        ],
        out_specs=pl.BlockSpec((_BLK, D), lambda i: (i, 0)),
        out_shape=jax.ShapeDtypeStruct((N, D), jnp.float32),
    )(x, num, den, bias)
    return y


# speedup vs baseline: 67.1839x; 64.4524x over previous
import jax
import jax.numpy as jnp
from jax import lax
from jax.experimental import pallas as pl
from jax.experimental.pallas import tpu as pltpu
from jax.experimental.pallas import tpu_sc as plsc

N = 50000
E = 800000
H = 6
C = 16
D = H * C
W = 128
NEG = 0.2

_BLK = 2000

NC = 2
NS = 16
R16 = 6256
NROUND = 4
RPAD = 6264
TRASH = 6256
E2 = 819200
EPT = E2 // NS
CHUNK = 1024
CROWS = CHUNK // 128
NCHUNK = EPT // CHUNK
RING = 2048
RSTR = RING + 128
K = 128
ROWS_A = 392
PIECE = 56


def _proj_body(x_ref, wlT_ref, bl_ref, wrT_ref, br_ref, att_ref,
               xl_ref, xr_ref, init_ref):
    xb = x_ref[...]
    xl = xb @ wlT_ref[...] + bl_ref[...][None, :]
    xr = xb @ wrT_ref[...] + br_ref[...][None, :]
    zpad = jnp.zeros((_BLK, W - D), jnp.float32)
    xl_ref[...] = jnp.concatenate([xl, zpad], axis=1)
    xr_ref[...] = jnp.concatenate([xr, zpad], axis=1)
    xl3 = xl.reshape(_BLK, H, C)
    m = xl3 + xr.reshape(_BLK, H, C)
    m = jnp.where(m > 0, m, NEG * m)
    w = jnp.exp(jnp.sum(m * att_ref[...][None, :, :], axis=-1))
    num0 = (w[:, :, None] * xl3).reshape(_BLK, D)
    init_ref[...] = jnp.concatenate(
        [num0, w, jnp.zeros((_BLK, W - D - H), jnp.float32)], axis=1)


def _finish_body(x_ref, acc_ref, bias_ref, y_ref):
    acc = acc_ref[...]
    num = acc[:, :D].reshape(_BLK, H, C)
    den = acc[:, D:D + H]
    out = (num / den[:, :, None]).reshape(_BLK, D) + bias_ref[...][None, :]
    y_ref[...] = jnp.maximum(x_ref[...] + out, 0.0)


def _edge_kernel(xl_hbm, xr_hbm, src_hbm, dst_hbm, att_hbm, init_hbm,
                 out_hbm,
                 num_sh, src_ring, dstg_ring, dloc_ring, stage_src, stage_dst,
                 posbuf, dloc_stage, posw, padsrc, paddst, padloc,
                 xlbuf, xrbuf, outbuf, sidx, didx, dlidx, attv, cntbuf,
                 dsem):
    core = lax.axis_index("c")
    sid = lax.axis_index("s")

    pltpu.sync_copy(att_hbm, attv)
    att_v = [attv[h] for h in range(H)]
    iota = lax.iota(jnp.int32, 16)
    _ONES = jnp.ones((16,), jnp.int32)
    _ZEROS = jnp.zeros((16,), jnp.int32)
    _FIFTEEN = jnp.full((16,), 15, jnp.int32)
    eq = [iota == h for h in range(H)]
    SHIFT_IDX = [jnp.maximum(iota - d, 0) for d in (1, 2, 4, 8)]
    SHIFT_MSK = [iota >= d for d in (1, 2, 4, 8)]
    XOR_IDX = [iota ^ d for d in (1, 2, 4, 8)]

    padsrc[...] = _ZEROS
    paddst[...] = _ZEROS
    padloc[...] = iota - iota + TRASH
    rbase = sid * RSTR

    def _take(v, idx):
        return v.at[idx].get(mode="promise_in_bounds")

    def _to_scalar(fv):
        cntbuf[...] = fv
        return cntbuf[...][0]

    def _sumall(v):
        for idx in XOR_IDX:
            v = v + _take(v, idx)
        return v

    def _prefix(v):
        for idx, msk in zip(SHIFT_IDX, SHIFT_MSK):
            v = v + jnp.where(msk, _take(v, idx), _ZEROS)
        return v

    def staged_rows(hbm, hbm_off, sh_off, total, to_hbm):
        done = 0
        while done < total:
            sz = min(PIECE, total - done)
            stage = outbuf.at[pl.ds(0, sz)]
            if to_hbm:
                pltpu.sync_copy(num_sh.at[pl.ds(sh_off + done, sz)], stage)
                pltpu.sync_copy(stage, hbm.at[pl.ds(hbm_off + done, sz)])
            else:
                pltpu.sync_copy(hbm.at[pl.ds(hbm_off + done, sz)], stage)
                pltpu.sync_copy(stage, num_sh.at[pl.ds(sh_off + done, sz)])
            done += sz

    def drain_body(i, head):
        hb = pl.multiple_of(rbase + (head & (RING - 1)), 128)
        r1 = pltpu.async_copy(src_ring.at[pl.ds(hb, K)], sidx, dsem)
        r2 = pltpu.async_copy(dstg_ring.at[pl.ds(hb, K)], didx, dsem)
        r3 = pltpu.async_copy(dloc_ring.at[pl.ds(hb, K)], dlidx, dsem)
        r1.wait()
        r2.wait()
        r3.wait()
        c1 = pltpu.async_copy(xl_hbm.at[sidx], xlbuf, dsem)
        c2 = pltpu.async_copy(xr_hbm.at[didx], xrbuf, dsem)
        c1.wait()
        c2.wait()

        def edge(e, _):
            wvec = jnp.zeros((16,), jnp.float32)
            for h in range(H):
                a = xlbuf[e, pl.ds(h * C, C)]
                m = a + xrbuf[e, pl.ds(h * C, C)]
                m = jnp.maximum(m, NEG * m)
                bw = jnp.exp(_sumall(m * att_v[h]))
                outbuf[e, pl.ds(h * C, C)] = bw * a
                wvec = jnp.where(eq[h], bw, wvec)
            outbuf[e, pl.ds(D, 16)] = wvec
            return 0

        lax.fori_loop(0, K, edge, 0)
        pltpu.async_copy(outbuf, num_sh.at[dlidx], dsem, add=True).wait()
        return head + K

    def drain_upto(fill_s, head):
        nblocks = (fill_s - head) >> 7
        return lax.fori_loop(0, nblocks, drain_body, head)

    for r in range(NROUND):
        base = (core * NROUND + r) * R16
        upper = jnp.minimum(base + R16, jnp.int32(N))

        @pl.when(sid < 15)
        def _seed_a():
            staged_rows(init_hbm, base + sid * ROWS_A, sid * ROWS_A,
                        ROWS_A, to_hbm=False)

        if r < NROUND - 1:
            @pl.when(sid == 15)
            def _seed_b():
                staged_rows(init_hbm, base + 15 * ROWS_A, 15 * ROWS_A,
                            R16 - 15 * ROWS_A, to_hbm=False)
        else:
            @pl.when((sid == 15) & (core == 0))
            def _seed_b0():
                staged_rows(init_hbm, base + 15 * ROWS_A, 15 * ROWS_A,
                            R16 - 15 * ROWS_A, to_hbm=False)

            @pl.when((sid == 15) & (core == 1))
            def _seed_b1():
                staged_rows(init_hbm, base + 15 * ROWS_A, 15 * ROWS_A,
                            N - 7 * R16 - 15 * ROWS_A, to_hbm=False)
        plsc.subcore_barrier()

        def chunk_body(ch, carry):
            fill, head = carry
            offr = pl.multiple_of(sid * (EPT // 128) + ch * CROWS, 8)
            pltpu.sync_copy(src_hbm.at[pl.ds(offr, CROWS)], stage_src)
            pltpu.sync_copy(dst_hbm.at[pl.ds(offr, CROWS)], stage_dst)

            def scan_g(g, f):
                row = g >> 3
                csl = pl.ds((g & 7) * 16, 16)
                sv = stage_src[row, csl]
                dv = stage_dst[row, csl]
                inr = (dv >= base) & (dv < upper)
                mi = jnp.where(inr, _ONES, _ZEROS)
                cs = _prefix(mi)
                pos = rbase + jnp.where(inr, (f + cs - 1) & (RING - 1),
                                        RING + (g & 7) * 16 + iota)
                posbuf[row, csl] = pos
                dloc_stage[row, csl] = jnp.where(inr, dv - base,
                                                 iota - iota + TRASH)
                stage_dst[row, csl] = jnp.where(inr, dv, _ZEROS)
                return f + _take(cs, _FIFTEEN)

            fill = lax.fori_loop(0, CHUNK // 16, scan_g, fill)
            cps = []
            for rw in range(CROWS):
                ix = posbuf.at[rw]
                cps.append(pltpu.async_copy(stage_src.at[rw],
                                            src_ring.at[ix], dsem))
                cps.append(pltpu.async_copy(stage_dst.at[rw],
                                            dstg_ring.at[ix], dsem))
                cps.append(pltpu.async_copy(dloc_stage.at[rw],
                                            dloc_ring.at[ix], dsem))
            for c in cps:
                c.wait()
            head = drain_upto(_to_scalar(fill), head)
            return fill, head

        fill, head = lax.fori_loop(0, NCHUNK, chunk_body,
                                   (_ZEROS, jnp.int32(0)))

        fill_s = _to_scalar(fill)
        tgt = (fill_s + (K - 1)) & jnp.int32(-K)
        npad = (tgt - fill_s + 15) >> 4

        def pad_body(g, fv):
            posw[...] = rbase + ((fv + iota) & (RING - 1))
            p1 = pltpu.async_copy(padsrc, src_ring.at[posw], dsem)
            p2 = pltpu.async_copy(paddst, dstg_ring.at[posw], dsem)
            p3 = pltpu.async_copy(padloc, dloc_ring.at[posw], dsem)
            p1.wait()
            p2.wait()
            p3.wait()
            return fv + 16

        lax.fori_loop(0, npad, pad_body, fill)
        head = drain_upto(tgt, head)

        plsc.subcore_barrier()

        @pl.when(sid < 15)
        def _wb_a():
            staged_rows(out_hbm, base + sid * ROWS_A, sid * ROWS_A,
                        ROWS_A, to_hbm=True)

        if r < NROUND - 1:
            @pl.when(sid == 15)
            def _wb_b():
                staged_rows(out_hbm, base + 15 * ROWS_A, 15 * ROWS_A,
                            R16 - 15 * ROWS_A, to_hbm=True)
        else:
            @pl.when((sid == 15) & (core == 0))
            def _wb_b0():
                staged_rows(out_hbm, base + 15 * ROWS_A, 15 * ROWS_A,
                            R16 - 15 * ROWS_A, to_hbm=True)

            @pl.when((sid == 15) & (core == 1))
            def _wb_b1():
                staged_rows(out_hbm, base + 15 * ROWS_A, 15 * ROWS_A,
                            N - 7 * R16 - 15 * ROWS_A, to_hbm=True)
        plsc.subcore_barrier()


def kernel(x, edge_index, Wl, bl, Wr, br, att, bias):
    grid = N // _BLK
    xl, xr, init = pl.pallas_call(
        _proj_body,
        grid=(grid,),
        in_specs=[
            pl.BlockSpec((_BLK, D), lambda i: (i, 0)),
            pl.BlockSpec((D, D), lambda i: (0, 0)),
            pl.BlockSpec((D,), lambda i: (0,)),
            pl.BlockSpec((D, D), lambda i: (0, 0)),
            pl.BlockSpec((D,), lambda i: (0,)),
            pl.BlockSpec((H, C), lambda i: (0, 0)),
        ],
        out_specs=[
            pl.BlockSpec((_BLK, W), lambda i: (i, 0)),
            pl.BlockSpec((_BLK, W), lambda i: (i, 0)),
            pl.BlockSpec((_BLK, W), lambda i: (i, 0)),
        ],
        out_shape=[
            jax.ShapeDtypeStruct((N, W), jnp.float32),
            jax.ShapeDtypeStruct((N, W), jnp.float32),
            jax.ShapeDtypeStruct((N, W), jnp.float32),
        ],
    )(x, Wl.T, bl, Wr.T, br, att.reshape(H, C))

    npad_e = E2 - E
    src2 = jnp.concatenate(
        [edge_index[0], jnp.zeros((npad_e,), jnp.int32)]).reshape(E2 // 128, 128)
    dst2 = jnp.concatenate(
        [edge_index[1], jnp.full((npad_e,), N, jnp.int32)]).reshape(E2 // 128, 128)
    mesh = plsc.VectorSubcoreMesh(core_axis_name="c", subcore_axis_name="s",
                                  num_cores=NC, num_subcores=NS)
    acc = pl.kernel(
        _edge_kernel,
        out_type=jax.ShapeDtypeStruct((N, W), jnp.float32),
        mesh=mesh,
        scratch_types=[
            pltpu.VMEM_SHARED((RPAD, W), jnp.float32),
            pltpu.VMEM_SHARED((NS * RSTR,), jnp.int32),
            pltpu.VMEM_SHARED((NS * RSTR,), jnp.int32),
            pltpu.VMEM_SHARED((NS * RSTR,), jnp.int32),
            pltpu.VMEM((CROWS, 128), jnp.int32),
            pltpu.VMEM((CROWS, 128), jnp.int32),
            pltpu.VMEM((CROWS, 128), jnp.int32),
            pltpu.VMEM((CROWS, 128), jnp.int32),
            pltpu.VMEM((16,), jnp.int32),
            pltpu.VMEM((16,), jnp.int32),
            pltpu.VMEM((16,), jnp.int32),
            pltpu.VMEM((16,), jnp.int32),
            pltpu.VMEM((K, W), jnp.float32),
            pltpu.VMEM((K, W), jnp.float32),
            pltpu.VMEM((K, W), jnp.float32),
            pltpu.VMEM((K,), jnp.int32),
            pltpu.VMEM((K,), jnp.int32),
            pltpu.VMEM((K,), jnp.int32),
            pltpu.VMEM((H, C), jnp.float32),
            pltpu.VMEM((16,), jnp.int32),
            pltpu.SemaphoreType.DMA,
        ],
    )(xl, xr, src2, dst2, att.reshape(H, C), init)

    y = pl.pallas_call(
        _finish_body,
        grid=(grid,),
        in_specs=[
            pl.BlockSpec((_BLK, D), lambda i: (i, 0)),
            pl.BlockSpec((_BLK, W), lambda i: (i, 0)),
            pl.BlockSpec((D,), lambda i: (0,)),
        ],
        out_specs=pl.BlockSpec((_BLK, D), lambda i: (i, 0)),
        out_shape=jax.ShapeDtypeStruct((N, D), jnp.float32),
    )(x, acc, bias)
    return y
